# Initial kernel scaffold; baseline (speedup 1.0000x reference)
import jax
import jax.numpy as jnp
from jax.experimental import pallas as pl


def kernel(x, W, b, codebook):
    raise NotImplementedError("write your pallas kernel here")


# speedup vs baseline: 6.6290x; 6.6290x over previous
import jax
import jax.numpy as jnp
from jax.experimental import pallas as pl

_GROUPS = 2
_NUM_VARS = 512
_VAR_DIM = 64
_N_TILE = 256


def _vq_kernel(x_ref, w_ref, b_ref, cb_ref, out_ref, probs_ref):
    i = pl.program_id(0)
    logits = jax.lax.dot_general(
        x_ref[:], w_ref[:],
        dimension_numbers=(((1,), (1,)), ((), ())),
        preferred_element_type=jnp.float32,
    ) + b_ref[:]
    psums = []
    for g in range(_GROUPS):
        lg = logits[:, g * _NUM_VARS:(g + 1) * _NUM_VARS]
        m = jnp.max(lg, axis=-1, keepdims=True)
        e = jnp.exp(lg - m)
        s = jnp.sum(e, axis=-1, keepdims=True)
        psums.append(jnp.sum(e / s, axis=0))
        idx = jax.lax.broadcasted_iota(jnp.int32, lg.shape, 1)
        k = jnp.min(jnp.where(lg == m, idx, _NUM_VARS), axis=-1, keepdims=True)
        onehot = (idx == k).astype(jnp.float32)
        outg = jax.lax.dot_general(
            onehot, cb_ref[g * _NUM_VARS:(g + 1) * _NUM_VARS, :],
            dimension_numbers=(((1,), (0,)), ((), ())),
            preferred_element_type=jnp.float32,
        )
        out_ref[:, g * _VAR_DIM:(g + 1) * _VAR_DIM] = outg
    psum = jnp.concatenate(psums).reshape(1, _GROUPS * _NUM_VARS)

    @pl.when(i == 0)
    def _():
        probs_ref[:] = psum

    @pl.when(i != 0)
    def _():
        probs_ref[:] = probs_ref[:] + psum


def kernel(x, W, b, codebook):
    bsz, t, d = x.shape
    n = bsz * t
    flat = x.reshape(n, d)
    cb = codebook.reshape(_GROUPS * _NUM_VARS, _VAR_DIM)
    grid = n // _N_TILE
    out, probs = pl.pallas_call(
        _vq_kernel,
        grid=(grid,),
        in_specs=[
            pl.BlockSpec((_N_TILE, d), lambda i: (i, 0)),
            pl.BlockSpec((_GROUPS * _NUM_VARS, d), lambda i: (0, 0)),
            pl.BlockSpec((1, _GROUPS * _NUM_VARS), lambda i: (0, 0)),
            pl.BlockSpec((_GROUPS * _NUM_VARS, _VAR_DIM), lambda i: (0, 0)),
        ],
        out_specs=[
            pl.BlockSpec((_N_TILE, _GROUPS * _VAR_DIM), lambda i: (i, 0)),
            pl.BlockSpec((1, _GROUPS * _NUM_VARS), lambda i: (0, 0)),
        ],
        out_shape=[
            jax.ShapeDtypeStruct((n, _GROUPS * _VAR_DIM), jnp.float32),
            jax.ShapeDtypeStruct((1, _GROUPS * _NUM_VARS), jnp.float32),
        ],
    )(flat, W, b.reshape(1, -1), cb)
    avg_probs = (probs / n).reshape(_GROUPS, _NUM_VARS)
    return out.reshape(bsz, t, _GROUPS * _VAR_DIM), avg_probs
